# baseline (speedup 1.0000x reference)
import functools

import jax
import jax.numpy as jnp
from jax import lax
from jax.experimental import pallas as pl
from jax.experimental.pallas import tpu as pltpu
from jax.experimental.pallas import tpu_sc as plsc

_N = 10000
_NP = 10240
_E = 320000
_EP = 327680
_ER = _EP // 128
_D = 128
_T = 3
_NC = 2
_NS = 16
_RPT = _ER // (_NC * _NS)
_HALF = _ER // _NC
_RSUB = _NP // _NS
_NCLS = 20
_BLK = 2048
_GRID = _NP // _BLK
_ETS = ("cites", "similar", "coword")

_MESH = plsc.VectorSubcoreMesh(core_axis_name="c", subcore_axis_name="s")


def _deg_body(s0, s1, s2, d0, d1, d2, out, acc, idxbuf, onesb, ssem):
    c = lax.axis_index("c")
    s = lax.axis_index("s")
    zero16 = jnp.zeros((16,), jnp.float32)
    one16 = jnp.ones((16,), jnp.float32)

    def fill(band):
        def f(i, carry):
            for q in range(8):
                onesb[i, pl.ds(q * 16, 16)] = one16 if q == band else zero16
            return carry

        lax.fori_loop(0, 128, f, 0)

    fill(-1)
    for k in range(_RSUB // 128):
        pltpu.sync_copy(onesb, acc.at[pl.ds(s * _RSUB + k * 128, 128), :])
    plsc.subcore_barrier()

    row0 = c * _HALF + s * _RPT
    for d, earrs in enumerate(((s0, s1, s2), (d0, d1, d2))):
        for t in range(_T):
            fill(d * _T + t)
            pltpu.sync_copy(earrs[t].at[pl.ds(row0, _RPT), :], idxbuf)

            for bb in range(_RPT // 16):
                def fire(j, carry, bb=bb):
                    pltpu.async_copy(onesb, acc.at[idxbuf.at[bb * 16 + j]],
                                     ssem, add=True)
                    return carry

                def drain(j, carry, bb=bb):
                    pltpu.make_async_copy(
                        onesb, acc.at[idxbuf.at[bb * 16 + j]], ssem).wait()
                    return carry

                lax.fori_loop(0, 16, fire, 0)
                lax.fori_loop(0, 16, drain, 0)
    plsc.subcore_barrier()
    pltpu.sync_copy(acc.at[pl.ds(s * _RSUB, _RSUB), :],
                    out.at[c, pl.ds(s * _RSUB, _RSUB), :])


_deg_call = pl.kernel(
    _deg_body,
    out_type=jax.ShapeDtypeStruct((_NC, _NP, _D), jnp.float32),
    mesh=_MESH,
    scratch_types=(
        [pltpu.VMEM_SHARED((_NP, _D), jnp.float32),
         pltpu.VMEM((_RPT, 128), jnp.int32),
         pltpu.VMEM((128, _D), jnp.float32),
         pltpu.SemaphoreType.DMA]),
)


_RPS = _RPT


_ICH = 16


def _seg_body(p_hbm, es0, es1, es2, ed0, ed1, ed2, out,
              acc, r0, r1, sidxa, didxa, sidxb, didxb,
              gsem0, gsem1, ssem0, ssem1, isema, isemb):
    c = lax.axis_index("c")
    s = lax.axis_index("s")
    zero16 = jnp.zeros((16,), jnp.float32)

    def fz(i, carry):
        for q in range(_D // 16):
            r0[i, pl.ds(q * 16, 16)] = zero16
        return carry

    idxbufs = ((sidxa, didxa, isema), (sidxb, didxb, isemb))
    nch = _RPS // _ICH

    esrc = (es0, es1, es2)
    edst = (ed0, ed1, ed2)

    def idx_start(t, ch, pair):
        sb, db, isem = idxbufs[pair]
        base = c * _HALF + s * _RPS + ch * _ICH
        pltpu.async_copy(esrc[t].at[pl.ds(base, _ICH), :], sb, isem)
        pltpu.async_copy(edst[t].at[pl.ds(base, _ICH), :], db, isem)

    def idx_wait(t, ch, pair):
        sb, db, isem = idxbufs[pair]
        base = c * _HALF + s * _RPS + ch * _ICH
        pltpu.make_async_copy(esrc[t].at[pl.ds(base, _ICH), :], sb,
                              isem).wait()
        pltpu.make_async_copy(edst[t].at[pl.ds(base, _ICH), :], db,
                              isem).wait()

    for t in range(_T):
        lax.fori_loop(0, 128, fz, 0)
        for k in range(_RSUB // 128):
            pltpu.sync_copy(r0, acc.at[pl.ds(s * _RSUB + k * 128, 128), :])
        plsc.subcore_barrier()

        ptab = p_hbm.at[t]
        idx_start(t, 0, 0)
        idx_wait(t, 0, 0)
        pltpu.async_copy(ptab.at[idxbufs[0][0].at[0]], r0, gsem0)
        for ch in range(nch):
            sb, db, _ = idxbufs[ch % 2]
            if ch + 1 < nch:
                idx_start(t, ch + 1, (ch + 1) % 2)

            def body(k2, carry, ptab=ptab, sb=sb, db=db):
                j0 = 2 * k2
                j1 = j0 + 1
                pltpu.make_async_copy(ptab.at[sb.at[j0]], r0, gsem0).wait()
                pltpu.async_copy(r0, acc.at[db.at[j0]], ssem0, add=True)

                @pl.when(j0 > 0)
                def _(db=db, j0=j0):
                    pltpu.make_async_copy(r1, acc.at[db.at[j0 - 1]],
                                          ssem1).wait()

                pltpu.async_copy(ptab.at[sb.at[j1]], r1, gsem1)
                pltpu.make_async_copy(ptab.at[sb.at[j1]], r1, gsem1).wait()
                pltpu.async_copy(r1, acc.at[db.at[j1]], ssem1, add=True)
                pltpu.make_async_copy(r0, acc.at[db.at[j0]], ssem0).wait()
                pltpu.async_copy(ptab.at[sb.at[j1 + 1]], r0, gsem0)
                return carry

            lax.fori_loop(0, _ICH // 2 - 1, body, 0)

            j0, j1 = _ICH - 2, _ICH - 1
            if ch + 1 < nch:
                idx_wait(t, ch + 1, (ch + 1) % 2)
            pltpu.make_async_copy(ptab.at[sb.at[j0]], r0, gsem0).wait()
            pltpu.async_copy(r0, acc.at[db.at[j0]], ssem0, add=True)
            pltpu.make_async_copy(r1, acc.at[db.at[j0 - 1]], ssem1).wait()
            pltpu.async_copy(ptab.at[sb.at[j1]], r1, gsem1)
            pltpu.make_async_copy(ptab.at[sb.at[j1]], r1, gsem1).wait()
            pltpu.async_copy(r1, acc.at[db.at[j1]], ssem1, add=True)
            pltpu.make_async_copy(r0, acc.at[db.at[j0]], ssem0).wait()
            if ch + 1 < nch:
                nsb = idxbufs[(ch + 1) % 2][0]
                pltpu.async_copy(ptab.at[nsb.at[0]], r0, gsem0)
            pltpu.make_async_copy(r1, acc.at[db.at[j1]], ssem1).wait()
        plsc.subcore_barrier()

        pltpu.sync_copy(acc.at[pl.ds(s * _RSUB, _RSUB), :],
                        out.at[t, c, pl.ds(s * _RSUB, _RSUB), :])


_seg_call = pl.kernel(
    _seg_body,
    out_type=jax.ShapeDtypeStruct((_T, _NC, _NP, _D), jnp.float32),
    mesh=_MESH,
    scratch_types=(
        [pltpu.VMEM_SHARED((_NP, _D), jnp.float32),
         pltpu.VMEM((128, _D), jnp.float32),
         pltpu.VMEM((128, _D), jnp.float32),
         pltpu.VMEM((_ICH, 128), jnp.int32),
         pltpu.VMEM((_ICH, 128), jnp.int32),
         pltpu.VMEM((_ICH, 128), jnp.int32),
         pltpu.VMEM((_ICH, 128), jnp.int32)]
        + [pltpu.SemaphoreType.DMA] * 6),
)


def _mm3_body(h_ref, w_ref, cs_ref, o_ref):
    h = h_ref[...]
    cnt = cs_ref[0] + cs_ref[1]
    for t in range(_T):
        scale = lax.rsqrt(jnp.clip(cnt[:, 16 * t:16 * t + 1], 1.0, None))
        p = jnp.dot(h, w_ref[t], preferred_element_type=jnp.float32) * scale
        o_ref[t] = p


def _mm3(h, w3, cs):
    return pl.pallas_call(
        _mm3_body,
        grid=(_GRID,),
        in_specs=[
            pl.BlockSpec((_BLK, _D), lambda i: (i, 0)),
            pl.BlockSpec((_T, _D, _D), lambda i: (0, 0, 0)),
            pl.BlockSpec((_NC, _BLK, _D), lambda i: (0, i, 0)),
        ],
        out_specs=pl.BlockSpec((_T, _BLK, _D), lambda i: (0, i, 0)),
        out_shape=jax.ShapeDtypeStruct((_T, _NP, _D), jnp.float32),
    )(h, w3, cs)


def _combine_body(final, agg_ref, cd_ref, b_ref, a_ref, *refs):
    if final:
        wfc_ref, bfc_ref, h_out, log_out = refs
    else:
        (h_out,) = refs
    cnt = cd_ref[0] + cd_ref[1]
    b = b_ref[...]
    a = a_ref[...]
    a1 = a[0:_D].reshape(1, _D)
    a2 = a[_D:2 * _D].reshape(1, _D)
    outs = []
    for t in range(_T):
        g = agg_ref[t, 0] + agg_ref[t, 1]
        col = 16 * (_T + t)
        scale = lax.rsqrt(jnp.clip(cnt[:, col:col + 1], 1.0, None))
        outs.append(g * scale + b[t].reshape(1, _D))
    mean = (outs[0] + outs[1] + outs[2]) * (1.0 / 3.0)
    s2 = jnp.sum(mean * a2, axis=1, keepdims=True)
    ss = []
    for t in range(_T):
        v = jnp.sum(outs[t] * a1, axis=1, keepdims=True) + s2
        ss.append(jnp.where(v >= 0, v, 0.2 * v))
    m = jnp.maximum(jnp.maximum(ss[0], ss[1]), ss[2])
    es = [jnp.exp(v - m) for v in ss]
    den = es[0] + es[1] + es[2]
    h = (es[0] * outs[0] + es[1] * outs[1] + es[2] * outs[2]) / den
    if final:
        h_out[...] = h
        log_out[...] = (jnp.dot(h, wfc_ref[...],
                                preferred_element_type=jnp.float32)
                        + bfc_ref[...])[:, :_NCLS]
    else:
        h_out[...] = jnp.where(h >= 0, h, 0.01 * h)


def _combine(agg, cd, b3, a, final, wfc=None, bfc=None):
    blk = 2000 if final else _BLK
    outs = [jax.ShapeDtypeStruct((_N if final else _NP, _D), jnp.float32)]
    out_specs = [pl.BlockSpec((blk, _D), lambda i: (i, 0))]
    in_specs = [
        pl.BlockSpec((_T, _NC, blk, _D), lambda i: (0, 0, i, 0)),
        pl.BlockSpec((_NC, blk, _D), lambda i: (0, i, 0)),
        pl.BlockSpec((_T, _D), lambda i: (0, 0)),
        pl.BlockSpec((2 * _D,), lambda i: (0,)),
    ]
    args = [agg, cd, b3, a]
    if final:
        in_specs += [pl.BlockSpec((_D, _D), lambda i: (0, 0)),
                     pl.BlockSpec((1, _D), lambda i: (0, 0))]
        args += [wfc, bfc]
        outs.append(jax.ShapeDtypeStruct((_N, _NCLS), jnp.float32))
        out_specs.append(pl.BlockSpec((blk, _NCLS), lambda i: (i, 0)))
    return pl.pallas_call(
        functools.partial(_combine_body, final),
        grid=(_GRID,),
        in_specs=in_specs,
        out_specs=out_specs,
        out_shape=outs,
    )(*args)


def kernel(x, edge_index_cites, edge_index_similar, edge_index_coword, params):
    h = jnp.pad(x.astype(jnp.float32), ((0, _NP - _N), (0, 0)))
    srcs, dsts = [], []
    pad = _N + (jnp.arange(_EP - _E, dtype=jnp.int32) % (_NP - _N))
    for ei in (edge_index_cites, edge_index_similar, edge_index_coword):
        srcs.append(jnp.concatenate([ei[0].astype(jnp.int32), pad])
                    .reshape(_ER, 128))
        dsts.append(jnp.concatenate([ei[1].astype(jnp.int32), pad])
                    .reshape(_ER, 128))
    cnt = _deg_call(*srcs, *dsts)

    logits_p = None
    for l in range(2):
        w3 = jnp.stack([params[f"W{l}_{t}"] for t in _ETS])
        b3 = jnp.stack([params[f"b{l}_{t}"] for t in _ETS])
        p = _mm3(h, w3, cnt)
        agg = _seg_call(p, *srcs, *dsts)
        if l == 0:
            h = _combine(agg, cnt, b3, params["a_type0"], final=False)[0]
        else:
            wfc = jnp.zeros((_D, _D), jnp.float32).at[:, :_NCLS].set(
                params["W_fc"])
            bfc = jnp.zeros((1, _D), jnp.float32).at[0, :_NCLS].set(
                params["b_fc"])
            h, logits_p = _combine(agg, cnt, b3, params["a_type1"],
                                   final=True, wfc=wfc, bfc=bfc)
    return (h, logits_p)


# speedup vs baseline: 1.0071x; 1.0071x over previous
import functools

import jax
import jax.numpy as jnp
from jax import lax
from jax.experimental import pallas as pl
from jax.experimental.pallas import tpu as pltpu
from jax.experimental.pallas import tpu_sc as plsc

_N = 10000
_NP = 10240
_E = 320000
_EP = 327680
_ER = _EP // 128
_D = 128
_T = 3
_NC = 2
_NS = 16
_RPT = _ER // (_NC * _NS)
_HALF = _ER // _NC
_RSUB = _NP // _NS
_NCLS = 20
_BLK = 2048
_GRID = _NP // _BLK
_ETS = ("cites", "similar", "coword")

_MESH = plsc.VectorSubcoreMesh(core_axis_name="c", subcore_axis_name="s")


def _deg_body(s0, s1, s2, d0, d1, d2, out, acc, idxbuf, onesb, ssem):
    c = lax.axis_index("c")
    s = lax.axis_index("s")
    zero16 = jnp.zeros((16,), jnp.float32)
    one16 = jnp.ones((16,), jnp.float32)

    def fill(band):
        def f(i, carry):
            for q in range(8):
                onesb[i, pl.ds(q * 16, 16)] = one16 if q == band else zero16
            return carry

        lax.fori_loop(0, 128, f, 0)

    fill(-1)
    for k in range(_RSUB // 128):
        pltpu.sync_copy(onesb, acc.at[pl.ds(s * _RSUB + k * 128, 128), :])
    plsc.subcore_barrier()

    row0 = c * _HALF + s * _RPT
    for d, earrs in enumerate(((s0, s1, s2), (d0, d1, d2))):
        for t in range(_T):
            fill(d * _T + t)
            pltpu.sync_copy(earrs[t].at[pl.ds(row0, _RPT), :], idxbuf)

            for bb in range(_RPT // 16):
                def fire(j, carry, bb=bb):
                    pltpu.async_copy(onesb, acc.at[idxbuf.at[bb * 16 + j]],
                                     ssem, add=True)
                    return carry

                def drain(j, carry, bb=bb):
                    pltpu.make_async_copy(
                        onesb, acc.at[idxbuf.at[bb * 16 + j]], ssem).wait()
                    return carry

                lax.fori_loop(0, 16, fire, 0)
                lax.fori_loop(0, 16, drain, 0)
    plsc.subcore_barrier()
    pltpu.sync_copy(acc.at[pl.ds(s * _RSUB, _RSUB), :],
                    out.at[c, pl.ds(s * _RSUB, _RSUB), :])


_deg_call = pl.kernel(
    _deg_body,
    out_type=jax.ShapeDtypeStruct((_NC, _NP, _D), jnp.float32),
    mesh=_MESH,
    scratch_types=(
        [pltpu.VMEM_SHARED((_NP, _D), jnp.float32),
         pltpu.VMEM((_RPT, 128), jnp.int32),
         pltpu.VMEM((128, _D), jnp.float32),
         pltpu.SemaphoreType.DMA]),
)


_RPS = _RPT


_ICH = 16


def _seg_body(p_hbm, es0, es1, es2, ed0, ed1, ed2, out,
              acc, r0, r1, sidxa, didxa, sidxb, didxb,
              gsem0, gsem1, ssem0, ssem1, isema, isemb):
    c = lax.axis_index("c")
    s = lax.axis_index("s")
    zero16 = jnp.zeros((16,), jnp.float32)

    def fz(i, carry):
        for q in range(_D // 16):
            r0[i, pl.ds(q * 16, 16)] = zero16
        return carry

    idxbufs = ((sidxa, didxa, isema), (sidxb, didxb, isemb))
    nch = _RPS // _ICH

    esrc = (es0, es1, es2)
    edst = (ed0, ed1, ed2)

    def idx_start(t, ch, pair):
        sb, db, isem = idxbufs[pair]
        base = c * _HALF + s * _RPS + ch * _ICH
        pltpu.async_copy(esrc[t].at[pl.ds(base, _ICH), :], sb, isem)
        pltpu.async_copy(edst[t].at[pl.ds(base, _ICH), :], db, isem)

    def idx_wait(t, ch, pair):
        sb, db, isem = idxbufs[pair]
        base = c * _HALF + s * _RPS + ch * _ICH
        pltpu.make_async_copy(esrc[t].at[pl.ds(base, _ICH), :], sb,
                              isem).wait()
        pltpu.make_async_copy(edst[t].at[pl.ds(base, _ICH), :], db,
                              isem).wait()

    for t in range(_T):
        lax.fori_loop(0, 128, fz, 0)
        for k in range(_RSUB // 128):
            pltpu.sync_copy(r0, acc.at[pl.ds(s * _RSUB + k * 128, 128), :])
        plsc.subcore_barrier()

        ptab = p_hbm.at[t]
        idx_start(t, 0, 0)
        idx_wait(t, 0, 0)
        pltpu.async_copy(ptab.at[idxbufs[0][0].at[0]], r0, gsem0)
        for ch in range(nch):
            sb, db, _ = idxbufs[ch % 2]
            if ch + 1 < nch:
                idx_start(t, ch + 1, (ch + 1) % 2)

            def body(k2, carry, ptab=ptab, sb=sb, db=db):
                j0 = 2 * k2
                j1 = j0 + 1
                pltpu.make_async_copy(ptab.at[sb.at[j0]], r0, gsem0).wait()
                pltpu.async_copy(r0, acc.at[db.at[j0]], ssem0, add=True)

                @pl.when(j0 > 0)
                def _(db=db, j0=j0):
                    pltpu.make_async_copy(r1, acc.at[db.at[j0 - 1]],
                                          ssem1).wait()

                pltpu.async_copy(ptab.at[sb.at[j1]], r1, gsem1)
                pltpu.make_async_copy(ptab.at[sb.at[j1]], r1, gsem1).wait()
                pltpu.async_copy(r1, acc.at[db.at[j1]], ssem1, add=True)
                pltpu.make_async_copy(r0, acc.at[db.at[j0]], ssem0).wait()
                pltpu.async_copy(ptab.at[sb.at[j1 + 1]], r0, gsem0)
                return carry

            lax.fori_loop(0, _ICH // 2 - 1, body, 0)

            j0, j1 = _ICH - 2, _ICH - 1
            if ch + 1 < nch:
                idx_wait(t, ch + 1, (ch + 1) % 2)
            pltpu.make_async_copy(ptab.at[sb.at[j0]], r0, gsem0).wait()
            pltpu.async_copy(r0, acc.at[db.at[j0]], ssem0, add=True)
            pltpu.make_async_copy(r1, acc.at[db.at[j0 - 1]], ssem1).wait()
            pltpu.async_copy(ptab.at[sb.at[j1]], r1, gsem1)
            pltpu.make_async_copy(ptab.at[sb.at[j1]], r1, gsem1).wait()
            pltpu.async_copy(r1, acc.at[db.at[j1]], ssem1, add=True)
            pltpu.make_async_copy(r0, acc.at[db.at[j0]], ssem0).wait()
            if ch + 1 < nch:
                nsb = idxbufs[(ch + 1) % 2][0]
                pltpu.async_copy(ptab.at[nsb.at[0]], r0, gsem0)
            pltpu.make_async_copy(r1, acc.at[db.at[j1]], ssem1).wait()
        plsc.subcore_barrier()

        pltpu.sync_copy(acc.at[pl.ds(s * _RSUB, _RSUB), :],
                        out.at[t, c, pl.ds(s * _RSUB, _RSUB), :])


_seg_call = pl.kernel(
    _seg_body,
    out_type=jax.ShapeDtypeStruct((_T, _NC, _NP, _D), jnp.float32),
    mesh=_MESH,
    scratch_types=(
        [pltpu.VMEM_SHARED((_NP, _D), jnp.float32),
         pltpu.VMEM((128, _D), jnp.float32),
         pltpu.VMEM((128, _D), jnp.float32),
         pltpu.VMEM((_ICH, 128), jnp.int32),
         pltpu.VMEM((_ICH, 128), jnp.int32),
         pltpu.VMEM((_ICH, 128), jnp.int32),
         pltpu.VMEM((_ICH, 128), jnp.int32)]
        + [pltpu.SemaphoreType.DMA] * 6),
)


def _mm3_body(h_ref, w_ref, cs_ref, o_ref):
    h = h_ref[...]
    cnt = cs_ref[0] + cs_ref[1]
    for t in range(_T):
        scale = lax.rsqrt(jnp.clip(cnt[:, 16 * t:16 * t + 1], 1.0, None))
        p = jnp.dot(h, w_ref[t], preferred_element_type=jnp.float32) * scale
        o_ref[t] = p


def _mm3(h, w3, cs):
    return pl.pallas_call(
        _mm3_body,
        grid=(_GRID,),
        in_specs=[
            pl.BlockSpec((_BLK, _D), lambda i: (i, 0)),
            pl.BlockSpec((_T, _D, _D), lambda i: (0, 0, 0)),
            pl.BlockSpec((_NC, _BLK, _D), lambda i: (0, i, 0)),
        ],
        out_specs=pl.BlockSpec((_T, _BLK, _D), lambda i: (0, i, 0)),
        out_shape=jax.ShapeDtypeStruct((_T, _NP, _D), jnp.float32),
    )(h, w3, cs)


def _combine_body(final, agg_ref, cd_ref, b_ref, a_ref, *refs):
    if final:
        wfc_ref, bfc_ref, h_out, log_out = refs
    else:
        w_ref, p_out = refs
    cnt = cd_ref[0] + cd_ref[1]
    b = b_ref[...]
    a = a_ref[...]
    a1 = a[0:_D].reshape(1, _D)
    a2 = a[_D:2 * _D].reshape(1, _D)
    outs = []
    for t in range(_T):
        g = agg_ref[t, 0] + agg_ref[t, 1]
        col = 16 * (_T + t)
        scale = lax.rsqrt(jnp.clip(cnt[:, col:col + 1], 1.0, None))
        outs.append(g * scale + b[t].reshape(1, _D))
    mean = (outs[0] + outs[1] + outs[2]) * (1.0 / 3.0)
    s2 = jnp.sum(mean * a2, axis=1, keepdims=True)
    ss = []
    for t in range(_T):
        v = jnp.sum(outs[t] * a1, axis=1, keepdims=True) + s2
        ss.append(jnp.where(v >= 0, v, 0.2 * v))
    m = jnp.maximum(jnp.maximum(ss[0], ss[1]), ss[2])
    es = [jnp.exp(v - m) for v in ss]
    den = es[0] + es[1] + es[2]
    h = (es[0] * outs[0] + es[1] * outs[1] + es[2] * outs[2]) / den
    if final:
        h_out[...] = h
        log_out[...] = (jnp.dot(h, wfc_ref[...],
                                preferred_element_type=jnp.float32)
                        + bfc_ref[...])[:, :_NCLS]
    else:
        ha = jnp.where(h >= 0, h, 0.01 * h)
        for t in range(_T):
            scale = lax.rsqrt(jnp.clip(cnt[:, 16 * t:16 * t + 1], 1.0, None))
            p = jnp.dot(ha, w_ref[t],
                        preferred_element_type=jnp.float32) * scale
            p_out[t] = p


def _combine(agg, cd, b3, a, final, wfc=None, bfc=None, w3n=None):
    blk = 2000 if final else _BLK
    in_specs = [
        pl.BlockSpec((_T, _NC, blk, _D), lambda i: (0, 0, i, 0)),
        pl.BlockSpec((_NC, blk, _D), lambda i: (0, i, 0)),
        pl.BlockSpec((_T, _D), lambda i: (0, 0)),
        pl.BlockSpec((2 * _D,), lambda i: (0,)),
    ]
    args = [agg, cd, b3, a]
    if final:
        in_specs += [pl.BlockSpec((_D, _D), lambda i: (0, 0)),
                     pl.BlockSpec((1, _D), lambda i: (0, 0))]
        args += [wfc, bfc]
        outs = [jax.ShapeDtypeStruct((_N, _D), jnp.float32),
                jax.ShapeDtypeStruct((_N, _NCLS), jnp.float32)]
        out_specs = [pl.BlockSpec((blk, _D), lambda i: (i, 0)),
                     pl.BlockSpec((blk, _NCLS), lambda i: (i, 0))]
    else:
        in_specs += [pl.BlockSpec((_T, _D, _D), lambda i: (0, 0, 0))]
        args += [w3n]
        outs = [jax.ShapeDtypeStruct((_T, _NP, _D), jnp.float32)]
        out_specs = [pl.BlockSpec((_T, blk, _D), lambda i: (0, i, 0))]
    return pl.pallas_call(
        functools.partial(_combine_body, final),
        grid=(_GRID,),
        in_specs=in_specs,
        out_specs=out_specs,
        out_shape=outs,
    )(*args)


def kernel(x, edge_index_cites, edge_index_similar, edge_index_coword, params):
    h = jnp.pad(x.astype(jnp.float32), ((0, _NP - _N), (0, 0)))
    srcs, dsts = [], []
    pad = _N + (jnp.arange(_EP - _E, dtype=jnp.int32) % (_NP - _N))
    for ei in (edge_index_cites, edge_index_similar, edge_index_coword):
        srcs.append(jnp.concatenate([ei[0].astype(jnp.int32), pad])
                    .reshape(_ER, 128))
        dsts.append(jnp.concatenate([ei[1].astype(jnp.int32), pad])
                    .reshape(_ER, 128))
    cnt = _deg_call(*srcs, *dsts)

    w3_0 = jnp.stack([params[f"W0_{t}"] for t in _ETS])
    b3_0 = jnp.stack([params[f"b0_{t}"] for t in _ETS])
    w3_1 = jnp.stack([params[f"W1_{t}"] for t in _ETS])
    b3_1 = jnp.stack([params[f"b1_{t}"] for t in _ETS])
    p0 = _mm3(h, w3_0, cnt)
    agg0 = _seg_call(p0, *srcs, *dsts)
    p1 = _combine(agg0, cnt, b3_0, params["a_type0"], final=False,
                  w3n=w3_1)[0]
    agg1 = _seg_call(p1, *srcs, *dsts)
    wfc = jnp.zeros((_D, _D), jnp.float32).at[:, :_NCLS].set(params["W_fc"])
    bfc = jnp.zeros((1, _D), jnp.float32).at[0, :_NCLS].set(params["b_fc"])
    h, logits_p = _combine(agg1, cnt, b3_1, params["a_type1"],
                           final=True, wfc=wfc, bfc=bfc)
    return (h, logits_p)
